# baseline (speedup 1.0000x reference)
import functools

import jax
import jax.numpy as jnp
from jax.experimental import pallas as pl
from jax.experimental.pallas import tpu as pltpu

N_NODES = 10000
D = 128
K_POOL = 1000
N_GRAPHS = 128

_ROW_BLK = 1000


def _gin_mlp_body(h_ref, a_ref, w1_ref, b1_ref, w2_ref, b2_ref, o_ref):
    z = h_ref[...] + a_ref[...]
    z = jnp.maximum(jnp.dot(z, w1_ref[...],
                            preferred_element_type=jnp.float32) + b1_ref[...], 0.0)
    z = jnp.dot(z, w2_ref[...], preferred_element_type=jnp.float32) + b2_ref[...]
    o_ref[...] = jnp.maximum(z, 0.0)


def _gin_mlp(h, agg, p):
    n = h.shape[0]
    blk = _ROW_BLK if n % _ROW_BLK == 0 else n
    return pl.pallas_call(
        _gin_mlp_body,
        grid=(n // blk,),
        in_specs=[
            pl.BlockSpec((blk, D), lambda i: (i, 0)),
            pl.BlockSpec((blk, D), lambda i: (i, 0)),
            pl.BlockSpec((D, D), lambda i: (0, 0)),
            pl.BlockSpec((1, D), lambda i: (0, 0)),
            pl.BlockSpec((D, D), lambda i: (0, 0)),
            pl.BlockSpec((1, D), lambda i: (0, 0)),
        ],
        out_specs=pl.BlockSpec((blk, D), lambda i: (i, 0)),
        out_shape=jax.ShapeDtypeStruct((n, D), jnp.float32),
    )(h, agg, p["W1"], p["b1"].reshape(1, D), p["W2"], p["b2"].reshape(1, D))


def _linear_body(x_ref, w_ref, b_ref, o_ref):
    o_ref[...] = jnp.dot(x_ref[...], w_ref[...],
                         preferred_element_type=jnp.float32) + b_ref[...]


def _linear(x, w, b):
    n, dout = x.shape[0], w.shape[1]
    blk = _ROW_BLK if n % _ROW_BLK == 0 else n
    return pl.pallas_call(
        _linear_body,
        grid=(n // blk,),
        in_specs=[
            pl.BlockSpec((blk, D), lambda i: (i, 0)),
            pl.BlockSpec((D, dout), lambda i: (0, 0)),
            pl.BlockSpec((1, dout), lambda i: (0, 0)),
        ],
        out_specs=pl.BlockSpec((blk, dout), lambda i: (i, 0)),
        out_shape=jax.ShapeDtypeStruct((n, dout), jnp.float32),
    )(x, w, b.reshape(1, dout))


def _stm_body(s_ref, m_ref, o_ref):
    @pl.when(pl.program_id(0) == 0)
    def _init():
        o_ref[...] = jnp.zeros_like(o_ref)

    o_ref[...] += jax.lax.dot_general(
        s_ref[...], m_ref[...], (((0,), (0,)), ((), ())),
        preferred_element_type=jnp.float32)


def _coarsen_matmul(S, M):
    n = S.shape[0]
    blk = _ROW_BLK
    C = pl.pallas_call(
        _stm_body,
        grid=(n // blk,),
        in_specs=[
            pl.BlockSpec((blk, K_POOL), lambda i: (i, 0)),
            pl.BlockSpec((blk, K_POOL), lambda i: (i, 0)),
        ],
        out_specs=pl.BlockSpec((K_POOL, K_POOL), lambda i: (0, 0)),
        out_shape=jax.ShapeDtypeStruct((K_POOL, K_POOL), jnp.float32),
    )(S, M)
    return C


def _pooled_gin_body(g_ref, c_ref, w1_ref, b1_ref, w2_ref, b2_ref, o_ref):
    g = g_ref[...]
    agg = jax.lax.dot_general(c_ref[...], g, (((0,), (0,)), ((), ())),
                              preferred_element_type=jnp.float32)
    z = g + agg
    z = jnp.maximum(jnp.dot(z, w1_ref[...],
                            preferred_element_type=jnp.float32) + b1_ref[...], 0.0)
    z = jnp.dot(z, w2_ref[...], preferred_element_type=jnp.float32) + b2_ref[...]
    o_ref[...] = jnp.maximum(z, 0.0)


def _pooled_gin(g, C, p):
    return pl.pallas_call(
        _pooled_gin_body,
        in_specs=[
            pl.BlockSpec((K_POOL, D), lambda: (0, 0)),
            pl.BlockSpec((K_POOL, K_POOL), lambda: (0, 0)),
            pl.BlockSpec((D, D), lambda: (0, 0)),
            pl.BlockSpec((1, D), lambda: (0, 0)),
            pl.BlockSpec((D, D), lambda: (0, 0)),
            pl.BlockSpec((1, D), lambda: (0, 0)),
        ],
        out_specs=pl.BlockSpec((K_POOL, D), lambda: (0, 0)),
        out_shape=jax.ShapeDtypeStruct((K_POOL, D), jnp.float32),
    )(g, C, p["W1"], p["b1"].reshape(1, D), p["W2"], p["b2"].reshape(1, D))


def kernel(x, edge_index, batch, sub_params, pool_params, gnn_params, cls_params):
    n = x.shape[0]
    src, dst = edge_index[0], edge_index[1]

    h = x
    for p in sub_params:
        agg = jax.ops.segment_sum(h[src], dst, num_segments=n)
        h = _gin_mlp(h, agg, p)

    loop = jnp.arange(n, dtype=src.dtype)
    row = jnp.concatenate([src, loop])
    col = jnp.concatenate([dst, loop])

    xq = jax.ops.segment_max(h[row], col, num_segments=n)
    xq_lin = _linear(xq, pool_params["lin_W"], pool_params["lin_b"])

    att_w1 = pool_params["att_W"][:D]
    att_w2 = pool_params["att_W"][D:]
    wpad = jnp.zeros((D, D), jnp.float32)
    wpad = wpad.at[:, 0:1].set(att_w1).at[:, 1:2].set(att_w2)
    qp = _linear(xq_lin, wpad, jnp.zeros((D,), jnp.float32))
    hp = _linear(h, wpad, jnp.zeros((D,), jnp.float32))
    qn = qp[:, 0]
    pn = hp[:, 1]

    score = qn[col] + pn[row] + pool_params["att_b"][0]
    score = jax.nn.leaky_relu(score, negative_slope=0.2)
    smax = jax.ops.segment_max(score, col, num_segments=n)
    score = jnp.exp(score - smax[col])
    ssum = jax.ops.segment_sum(score, col, num_segments=n)
    score = score / (ssum[col] + 1e-16)

    xn = jax.ops.segment_sum(h[row] * score[:, None], col, num_segments=n)

    lepad = jnp.zeros((D, D), jnp.float32)
    lepad = (lepad.at[:, 0:1].set(pool_params["le1_W"])
                  .at[:, 1:2].set(pool_params["le2_W"])
                  .at[:, 2:3].set(pool_params["le3_W"]))
    lebias = jnp.zeros((D,), jnp.float32)
    lebias = lebias.at[0].set(pool_params["le1_b"][0]).at[2].set(pool_params["le3_b"][0])
    le = _linear(xn, lepad, lebias)
    a_n, b_n, c_n = le[:, 0], le[:, 1], le[:, 2]
    agg = jax.ops.segment_sum(a_n[row] - b_n[col], col, num_segments=n)
    fitness = jax.nn.sigmoid(agg + c_n)

    fit_k, perm = jax.lax.top_k(fitness, K_POOL)
    xp = xn[perm] * fit_k[:, None]
    batch_p = batch[perm]

    pos = jnp.full((n,), -1, dtype=jnp.int32).at[perm].set(
        jnp.arange(K_POOL, dtype=jnp.int32))
    kcol = pos[col]
    valid = (kcol >= 0).astype(jnp.float32)
    kcol_safe = jnp.where(kcol >= 0, kcol, 0)
    S = jnp.zeros((n, K_POOL), jnp.float32).at[row, kcol_safe].add(score * valid)
    M = jnp.zeros((n, K_POOL), jnp.float32)
    CH = 17000
    ne = row.shape[0]
    for i in range(0, ne, CH):
        M = M.at[row[i:i + CH]].add(S[col[i:i + CH]])
    C = _coarsen_matmul(S, M)
    C = C * (1.0 - jnp.eye(K_POOL, dtype=jnp.float32))

    g = xp
    for p in gnn_params:
        g = _pooled_gin(g, C, p)

    sums = jnp.zeros((N_GRAPHS, D), jnp.float32).at[batch_p].add(g)
    cnt = jnp.zeros((N_GRAPHS,), jnp.float32).at[batch_p].add(1.0)
    out = sums / jnp.maximum(cnt, 1.0)[:, None]
    return out @ cls_params["W"] + cls_params["b"]


# speedup vs baseline: 1.3184x; 1.3184x over previous
import functools

import jax
import jax.numpy as jnp
from jax import lax
from jax.experimental import pallas as pl
from jax.experimental.pallas import tpu as pltpu
from jax.experimental.pallas import tpu_sc as plsc

N_NODES = 10000
D = 128
K_POOL = 1000
KP = 1024
N_GRAPHS = 128

_ACC = 10240
_TROWS = _ACC // 16
_DUMMY = 10100
_CH = 128


def _sc_body(nphase, nchunks, scaled, table, srcl, dstl, scalel, zrows, out,
             src_v, dst_v, scale_v, rows_v, acc, sem):
    cid = lax.axis_index("c")
    sid = lax.axis_index("s")
    eper = nchunks * _CH
    base = cid * (16 * eper) + sid * eper
    trow = pl.multiple_of(sid * _TROWS, 8)
    for p in range(nphase):
        pltpu.sync_copy(zrows, acc.at[pl.ds(trow, _TROWS)])
        plsc.subcore_barrier()
        for c in range(nchunks):
            off = pl.multiple_of(base + c * _CH, _CH)
            pltpu.sync_copy(srcl.at[pl.ds(off, _CH)], src_v)
            pltpu.sync_copy(dstl.at[pl.ds(off, _CH)], dst_v)
            pltpu.async_copy(table.at[p].at[src_v], rows_v, sem).wait()
            if scaled:
                pltpu.sync_copy(scalel.at[pl.ds(off, _CH)], scale_v)

                def mulrow(g, carry):
                    base16 = pl.multiple_of(g * 16, 16)
                    svec = scale_v[pl.ds(base16, 16)]
                    for i in range(16):
                        b = lax.gather(
                            svec, jnp.full((16, 1), i, jnp.int32),
                            lax.GatherDimensionNumbers(
                                offset_dims=(), collapsed_slice_dims=(0,),
                                start_index_map=(0,)),
                            (1,),
                            mode=lax.GatherScatterMode.PROMISE_IN_BOUNDS)
                        r = base16 + i
                        for j in range(8):
                            rows_v[r, pl.ds(j * 16, 16)] = (
                                rows_v[r, pl.ds(j * 16, 16)] * b)
                    return carry

                lax.fori_loop(0, _CH // 16, mulrow, 0)
            pltpu.sync_copy(rows_v, acc.at[dst_v], add=True)
        plsc.subcore_barrier()

        @pl.when(cid == 0)
        def _():
            pltpu.sync_copy(acc.at[pl.ds(trow, _TROWS)],
                            out.at[p, 0, pl.ds(trow, _TROWS)])

        @pl.when(cid == 1)
        def _():
            pltpu.sync_copy(acc.at[pl.ds(trow, _TROWS)],
                            out.at[p, 1, pl.ds(trow, _TROWS)])

        if p < nphase - 1:
            plsc.subcore_barrier()


def _sc_seg_sum(table, srcl, dstl, scalel=None):
    P = table.shape[0]
    EP = srcl.shape[0]
    nchunks = EP // (32 * _CH)
    scaled = scalel is not None
    if scalel is None:
        scalel = jnp.zeros((_CH,), jnp.float32)
    zrows = jnp.zeros((_TROWS, D), jnp.float32)
    mesh = plsc.VectorSubcoreMesh(core_axis_name="c", subcore_axis_name="s")
    f = pl.kernel(
        functools.partial(_sc_body, P, nchunks, scaled),
        out_type=jax.ShapeDtypeStruct((P, 2, _ACC, D), jnp.float32),
        mesh=mesh,
        scratch_types=[
            pltpu.VMEM((_CH,), jnp.int32),
            pltpu.VMEM((_CH,), jnp.int32),
            pltpu.VMEM((_CH,), jnp.float32),
            pltpu.VMEM((_CH, D), jnp.float32),
            pltpu.VMEM_SHARED((_ACC, D), jnp.float32),
            pltpu.SemaphoreType.DMA,
        ],
    )
    return f(table, srcl, dstl, scalel, zrows)


def _gin_mlp_body(h_ref, a_ref, w1_ref, b1_ref, w2_ref, b2_ref, o_ref):
    z = h_ref[...] + a_ref[0] + a_ref[1]
    z = jnp.maximum(jnp.dot(z, w1_ref[...],
                            preferred_element_type=jnp.float32) + b1_ref[...], 0.0)
    z = jnp.dot(z, w2_ref[...], preferred_element_type=jnp.float32) + b2_ref[...]
    o_ref[...] = jnp.maximum(z, 0.0)


def _gin_mlp(h, pacc, p):
    n = h.shape[0]
    blk = 1000
    return pl.pallas_call(
        _gin_mlp_body,
        grid=(n // blk,),
        in_specs=[
            pl.BlockSpec((blk, D), lambda i: (i, 0)),
            pl.BlockSpec((2, blk, D), lambda i: (0, i, 0)),
            pl.BlockSpec((D, D), lambda i: (0, 0)),
            pl.BlockSpec((1, D), lambda i: (0, 0)),
            pl.BlockSpec((D, D), lambda i: (0, 0)),
            pl.BlockSpec((1, D), lambda i: (0, 0)),
        ],
        out_specs=pl.BlockSpec((blk, D), lambda i: (i, 0)),
        out_shape=jax.ShapeDtypeStruct((n, D), jnp.float32),
    )(h, pacc, p["W1"], p["b1"].reshape(1, D), p["W2"], p["b2"].reshape(1, D))


def _linear_body(x_ref, w_ref, b_ref, o_ref):
    o_ref[...] = jnp.dot(x_ref[...], w_ref[...],
                         preferred_element_type=jnp.float32) + b_ref[...]


def _linear(x, w, b):
    n, dout = x.shape[0], w.shape[1]
    blk = 1000 if n % 1000 == 0 else n
    return pl.pallas_call(
        _linear_body,
        grid=(n // blk,),
        in_specs=[
            pl.BlockSpec((blk, D), lambda i: (i, 0)),
            pl.BlockSpec((D, dout), lambda i: (0, 0)),
            pl.BlockSpec((1, dout), lambda i: (0, 0)),
        ],
        out_specs=pl.BlockSpec((blk, dout), lambda i: (i, 0)),
        out_shape=jax.ShapeDtypeStruct((n, dout), jnp.float32),
    )(x, w, b.reshape(1, dout))


def _cmat_body(s_ref, m_ref, o_ref):
    @pl.when(pl.program_id(0) == 0)
    def _init():
        o_ref[...] = jnp.zeros_like(o_ref)

    s = s_ref[...]
    m = m_ref[...]
    for kb2 in range(8):
        msum = m[kb2, 0] + m[kb2, 1]
        prod = lax.dot_general(s, msum, (((1,), (0,)), ((), ())),
                               preferred_element_type=jnp.float32)
        o_ref[:, :, kb2, :] += prod


def _coarsen_matmul(S_r, Mp):
    blk = 1024
    C = pl.pallas_call(
        _cmat_body,
        grid=(_ACC // blk,),
        in_specs=[
            pl.BlockSpec((8, blk, D), lambda i: (0, i, 0)),
            pl.BlockSpec((8, 2, blk, D), lambda i: (0, 0, i, 0)),
        ],
        out_specs=pl.BlockSpec((8, D, 8, D), lambda i: (0, 0, 0, 0)),
        out_shape=jax.ShapeDtypeStruct((8, D, 8, D), jnp.float32),
    )(S_r, Mp)
    return C.reshape(KP, KP)


def _pooled_gin_body(g_ref, c_ref, w1_ref, b1_ref, w2_ref, b2_ref, o_ref):
    g = g_ref[...]
    agg = lax.dot_general(c_ref[...], g, (((0,), (0,)), ((), ())),
                          preferred_element_type=jnp.float32)
    z = g + agg
    z = jnp.maximum(jnp.dot(z, w1_ref[...],
                            preferred_element_type=jnp.float32) + b1_ref[...], 0.0)
    z = jnp.dot(z, w2_ref[...], preferred_element_type=jnp.float32) + b2_ref[...]
    o_ref[...] = jnp.maximum(z, 0.0)


def _pooled_gin(g, C, p):
    k = g.shape[0]
    return pl.pallas_call(
        _pooled_gin_body,
        in_specs=[
            pl.BlockSpec((k, D), lambda: (0, 0)),
            pl.BlockSpec((k, k), lambda: (0, 0)),
            pl.BlockSpec((D, D), lambda: (0, 0)),
            pl.BlockSpec((1, D), lambda: (0, 0)),
            pl.BlockSpec((D, D), lambda: (0, 0)),
            pl.BlockSpec((1, D), lambda: (0, 0)),
        ],
        out_specs=pl.BlockSpec((k, D), lambda: (0, 0)),
        out_shape=jax.ShapeDtypeStruct((k, D), jnp.float32),
    )(g, C, p["W1"], p["b1"].reshape(1, D), p["W2"], p["b2"].reshape(1, D))


def _pad_to(a, size, fill):
    return jnp.concatenate(
        [a, jnp.full((size - a.shape[0],), fill, a.dtype)])


def kernel(x, edge_index, batch, sub_params, pool_params, gnn_params, cls_params):
    n = x.shape[0]
    src, dst = edge_index[0], edge_index[1]
    ne_pad_gin = 163840
    ne_pad = 172032

    src_p = _pad_to(src, ne_pad_gin, 0)
    dst_p = _pad_to(dst, ne_pad_gin, _DUMMY)

    h = x
    for p in sub_params:
        pacc = _sc_seg_sum(h.reshape(1, n, D), src_p, dst_p)
        h = _gin_mlp(h, pacc[0, :, :n, :], p)

    loop = jnp.arange(n, dtype=src.dtype)
    row = jnp.concatenate([src, loop])
    col = jnp.concatenate([dst, loop])
    row_p = _pad_to(row, ne_pad, 0)
    col_p = _pad_to(col, ne_pad, _DUMMY)

    xq = jax.ops.segment_max(h[row], col, num_segments=n)
    xq_lin = _linear(xq, pool_params["lin_W"], pool_params["lin_b"])

    att_w1 = pool_params["att_W"][:D]
    att_w2 = pool_params["att_W"][D:]
    wpad = jnp.zeros((D, D), jnp.float32)
    wpad = wpad.at[:, 0:1].set(att_w1).at[:, 1:2].set(att_w2)
    qp = _linear(xq_lin, wpad, jnp.zeros((D,), jnp.float32))
    hp = _linear(h, wpad, jnp.zeros((D,), jnp.float32))
    qn = qp[:, 0]
    pn = hp[:, 1]

    score = qn[col] + pn[row] + pool_params["att_b"][0]
    score = jax.nn.leaky_relu(score, negative_slope=0.2)
    smax = jax.ops.segment_max(score, col, num_segments=n)
    score = jnp.exp(score - smax[col])
    ssum = jax.ops.segment_sum(score, col, num_segments=n)
    score = score / (ssum[col] + 1e-16)

    score_p = _pad_to(score, ne_pad, 0.0)
    xn_acc = _sc_seg_sum(h.reshape(1, n, D), row_p,
                         _pad_to(col, ne_pad, _DUMMY), score_p)
    xn = xn_acc[0, 0, :n, :] + xn_acc[0, 1, :n, :]

    lepad = jnp.zeros((D, D), jnp.float32)
    lepad = (lepad.at[:, 0:1].set(pool_params["le1_W"])
                  .at[:, 1:2].set(pool_params["le2_W"])
                  .at[:, 2:3].set(pool_params["le3_W"]))
    lebias = jnp.zeros((D,), jnp.float32)
    lebias = lebias.at[0].set(pool_params["le1_b"][0]).at[2].set(pool_params["le3_b"][0])
    le = _linear(xn, lepad, lebias)
    a_n, b_n, c_n = le[:, 0], le[:, 1], le[:, 2]
    agg = jax.ops.segment_sum(a_n[row] - b_n[col], col, num_segments=n)
    fitness = jax.nn.sigmoid(agg + c_n)

    fit_k, perm = jax.lax.top_k(fitness, K_POOL)
    batch_p = batch[perm]

    pos = jnp.full((n,), -1, dtype=jnp.int32).at[perm].set(
        jnp.arange(K_POOL, dtype=jnp.int32))
    kcol = pos[col]
    valid = (kcol >= 0).astype(jnp.float32)
    kcol_safe = jnp.where(kcol >= 0, kcol, 0)
    kb = kcol_safe // D
    kd = kcol_safe % D
    S_r = jnp.zeros((8, _ACC, D), jnp.float32).at[kb, row, kd].add(score * valid)
    Mp = _sc_seg_sum(S_r, col_p, row_p)
    C = _coarsen_matmul(S_r, Mp)
    C = C * (1.0 - jnp.eye(KP, dtype=jnp.float32))

    perm_pad = _pad_to(perm, KP, 0)
    fit_pad = _pad_to(fit_k, KP, 0.0)
    g = xn[perm_pad] * fit_pad[:, None]
    for p in gnn_params:
        g = _pooled_gin(g, C, p)
    g = g[:K_POOL]

    sums = jnp.zeros((N_GRAPHS, D), jnp.float32).at[batch_p].add(g)
    cnt = jnp.zeros((N_GRAPHS,), jnp.float32).at[batch_p].add(1.0)
    out = sums / jnp.maximum(cnt, 1.0)[:, None]
    return out @ cls_params["W"] + cls_params["b"]


# speedup vs baseline: 1.3822x; 1.0484x over previous
import functools

import jax
import jax.numpy as jnp
from jax import lax
from jax.experimental import pallas as pl
from jax.experimental.pallas import tpu as pltpu
from jax.experimental.pallas import tpu_sc as plsc

N_NODES = 10000
D = 128
K_POOL = 1000
KP = 1024
N_GRAPHS = 128

_ACC = 10240
_TROWS = _ACC // 16
_DUMMY = 10100
_CH = 128


def _sc_body(nphase, nchunks, scaled, table, srcl, dstl, scalel, zrows, out,
             src_v0, src_v1, dst_v, scale_v, rows_v0, rows_v1, acc,
             sem0, sem1):
    cid = lax.axis_index("c")
    sid = lax.axis_index("s")
    eper = nchunks * _CH
    base = cid * (16 * eper) + sid * eper
    trow = pl.multiple_of(sid * _TROWS, 8)
    src_v = [src_v0, src_v1]
    rows_v = [rows_v0, rows_v1]
    sem = [sem0, sem1]
    for p in range(nphase):
        pltpu.sync_copy(zrows, acc.at[pl.ds(trow, _TROWS)])
        plsc.subcore_barrier()
        descs = [None, None]
        pltpu.sync_copy(srcl.at[pl.ds(pl.multiple_of(base, _CH), _CH)],
                        src_v[0])
        descs[0] = pltpu.async_copy(table.at[p].at[src_v[0]], rows_v[0],
                                    sem[0])
        for c in range(nchunks):
            buf = c % 2
            off = pl.multiple_of(base + c * _CH, _CH)
            descs[buf].wait()
            if c + 1 < nchunks:
                nb = (c + 1) % 2
                noff = pl.multiple_of(base + (c + 1) * _CH, _CH)
                pltpu.sync_copy(srcl.at[pl.ds(noff, _CH)], src_v[nb])
                descs[nb] = pltpu.async_copy(table.at[p].at[src_v[nb]],
                                             rows_v[nb], sem[nb])
            rv = rows_v[buf]
            pltpu.sync_copy(dstl.at[pl.ds(off, _CH)], dst_v)
            if scaled:
                pltpu.sync_copy(scalel.at[pl.ds(off, _CH)], scale_v)

                def mulrow(g, carry):
                    base16 = pl.multiple_of(g * 16, 16)
                    svec = scale_v[pl.ds(base16, 16)]
                    for i in range(16):
                        b = lax.gather(
                            svec, jnp.full((16, 1), i, jnp.int32),
                            lax.GatherDimensionNumbers(
                                offset_dims=(), collapsed_slice_dims=(0,),
                                start_index_map=(0,)),
                            (1,),
                            mode=lax.GatherScatterMode.PROMISE_IN_BOUNDS)
                        r = base16 + i
                        for j in range(8):
                            rv[r, pl.ds(j * 16, 16)] = (
                                rv[r, pl.ds(j * 16, 16)] * b)
                    return carry

                lax.fori_loop(0, _CH // 16, mulrow, 0)
            pltpu.sync_copy(rv, acc.at[dst_v], add=True)
        plsc.subcore_barrier()

        @pl.when(cid == 0)
        def _():
            pltpu.sync_copy(acc.at[pl.ds(trow, _TROWS)],
                            out.at[p, 0, pl.ds(trow, _TROWS)])

        @pl.when(cid == 1)
        def _():
            pltpu.sync_copy(acc.at[pl.ds(trow, _TROWS)],
                            out.at[p, 1, pl.ds(trow, _TROWS)])

        if p < nphase - 1:
            plsc.subcore_barrier()


def _sc_seg_sum(table, srcl, dstl, scalel=None):
    P = table.shape[0]
    EP = srcl.shape[0]
    nchunks = EP // (32 * _CH)
    scaled = scalel is not None
    if scalel is None:
        scalel = jnp.zeros((_CH,), jnp.float32)
    zrows = jnp.zeros((_TROWS, D), jnp.float32)
    mesh = plsc.VectorSubcoreMesh(core_axis_name="c", subcore_axis_name="s")
    f = pl.kernel(
        functools.partial(_sc_body, P, nchunks, scaled),
        out_type=jax.ShapeDtypeStruct((P, 2, _ACC, D), jnp.float32),
        mesh=mesh,
        scratch_types=[
            pltpu.VMEM((_CH,), jnp.int32),
            pltpu.VMEM((_CH,), jnp.int32),
            pltpu.VMEM((_CH,), jnp.int32),
            pltpu.VMEM((_CH,), jnp.float32),
            pltpu.VMEM((_CH, D), jnp.float32),
            pltpu.VMEM((_CH, D), jnp.float32),
            pltpu.VMEM_SHARED((_ACC, D), jnp.float32),
            pltpu.SemaphoreType.DMA,
            pltpu.SemaphoreType.DMA,
        ],
    )
    return f(table, srcl, dstl, scalel, zrows)


def _gin_mlp_body(h_ref, a_ref, w1_ref, b1_ref, w2_ref, b2_ref, o_ref):
    z = h_ref[...] + a_ref[0] + a_ref[1]
    z = jnp.maximum(jnp.dot(z, w1_ref[...],
                            preferred_element_type=jnp.float32) + b1_ref[...], 0.0)
    z = jnp.dot(z, w2_ref[...], preferred_element_type=jnp.float32) + b2_ref[...]
    o_ref[...] = jnp.maximum(z, 0.0)


def _gin_mlp(h, pacc, p):
    n = h.shape[0]
    blk = 1000
    return pl.pallas_call(
        _gin_mlp_body,
        grid=(n // blk,),
        in_specs=[
            pl.BlockSpec((blk, D), lambda i: (i, 0)),
            pl.BlockSpec((2, blk, D), lambda i: (0, i, 0)),
            pl.BlockSpec((D, D), lambda i: (0, 0)),
            pl.BlockSpec((1, D), lambda i: (0, 0)),
            pl.BlockSpec((D, D), lambda i: (0, 0)),
            pl.BlockSpec((1, D), lambda i: (0, 0)),
        ],
        out_specs=pl.BlockSpec((blk, D), lambda i: (i, 0)),
        out_shape=jax.ShapeDtypeStruct((n, D), jnp.float32),
    )(h, pacc, p["W1"], p["b1"].reshape(1, D), p["W2"], p["b2"].reshape(1, D))


def _linear_body(x_ref, w_ref, b_ref, o_ref):
    o_ref[...] = jnp.dot(x_ref[...], w_ref[...],
                         preferred_element_type=jnp.float32) + b_ref[...]


def _linear(x, w, b):
    n, dout = x.shape[0], w.shape[1]
    blk = 1000 if n % 1000 == 0 else n
    return pl.pallas_call(
        _linear_body,
        grid=(n // blk,),
        in_specs=[
            pl.BlockSpec((blk, D), lambda i: (i, 0)),
            pl.BlockSpec((D, dout), lambda i: (0, 0)),
            pl.BlockSpec((1, dout), lambda i: (0, 0)),
        ],
        out_specs=pl.BlockSpec((blk, dout), lambda i: (i, 0)),
        out_shape=jax.ShapeDtypeStruct((n, dout), jnp.float32),
    )(x, w, b.reshape(1, dout))


def _cmat_body(s_ref, m_ref, o_ref):
    @pl.when(pl.program_id(0) == 0)
    def _init():
        o_ref[...] = jnp.zeros_like(o_ref)

    s = s_ref[...]
    m = m_ref[...]
    for kb2 in range(8):
        msum = m[kb2, 0] + m[kb2, 1]
        prod = lax.dot_general(s, msum, (((1,), (0,)), ((), ())),
                               preferred_element_type=jnp.float32)
        o_ref[:, :, kb2, :] += prod


def _coarsen_matmul(S_r, Mp):
    blk = 1024
    C = pl.pallas_call(
        _cmat_body,
        grid=(_ACC // blk,),
        in_specs=[
            pl.BlockSpec((8, blk, D), lambda i: (0, i, 0)),
            pl.BlockSpec((8, 2, blk, D), lambda i: (0, 0, i, 0)),
        ],
        out_specs=pl.BlockSpec((8, D, 8, D), lambda i: (0, 0, 0, 0)),
        out_shape=jax.ShapeDtypeStruct((8, D, 8, D), jnp.float32),
    )(S_r, Mp)
    return C.reshape(KP, KP)


def _pooled_gin_body(g_ref, c_ref, w1_ref, b1_ref, w2_ref, b2_ref, o_ref):
    g = g_ref[...]
    agg = lax.dot_general(c_ref[...], g, (((0,), (0,)), ((), ())),
                          preferred_element_type=jnp.float32)
    z = g + agg
    z = jnp.maximum(jnp.dot(z, w1_ref[...],
                            preferred_element_type=jnp.float32) + b1_ref[...], 0.0)
    z = jnp.dot(z, w2_ref[...], preferred_element_type=jnp.float32) + b2_ref[...]
    o_ref[...] = jnp.maximum(z, 0.0)


def _pooled_gin(g, C, p):
    k = g.shape[0]
    return pl.pallas_call(
        _pooled_gin_body,
        in_specs=[
            pl.BlockSpec((k, D), lambda: (0, 0)),
            pl.BlockSpec((k, k), lambda: (0, 0)),
            pl.BlockSpec((D, D), lambda: (0, 0)),
            pl.BlockSpec((1, D), lambda: (0, 0)),
            pl.BlockSpec((D, D), lambda: (0, 0)),
            pl.BlockSpec((1, D), lambda: (0, 0)),
        ],
        out_specs=pl.BlockSpec((k, D), lambda: (0, 0)),
        out_shape=jax.ShapeDtypeStruct((k, D), jnp.float32),
    )(g, C, p["W1"], p["b1"].reshape(1, D), p["W2"], p["b2"].reshape(1, D))


def _pad_to(a, size, fill):
    return jnp.concatenate(
        [a, jnp.full((size - a.shape[0],), fill, a.dtype)])


def kernel(x, edge_index, batch, sub_params, pool_params, gnn_params, cls_params):
    n = x.shape[0]
    src, dst = edge_index[0], edge_index[1]
    ne_pad_gin = 163840
    ne_pad = 172032

    src_p = _pad_to(src, ne_pad_gin, 0)
    dst_p = _pad_to(dst, ne_pad_gin, _DUMMY)

    h = x
    for p in sub_params:
        pacc = _sc_seg_sum(h.reshape(1, n, D), src_p, dst_p)
        h = _gin_mlp(h, pacc[0, :, :n, :], p)

    loop = jnp.arange(n, dtype=src.dtype)
    row = jnp.concatenate([src, loop])
    col = jnp.concatenate([dst, loop])
    row_p = _pad_to(row, ne_pad, 0)
    col_p = _pad_to(col, ne_pad, _DUMMY)

    xq = jax.ops.segment_max(h[row], col, num_segments=n)
    xq_lin = _linear(xq, pool_params["lin_W"], pool_params["lin_b"])

    att_w1 = pool_params["att_W"][:D]
    att_w2 = pool_params["att_W"][D:]
    wpad = jnp.zeros((D, D), jnp.float32)
    wpad = wpad.at[:, 0:1].set(att_w1).at[:, 1:2].set(att_w2)
    qp = _linear(xq_lin, wpad, jnp.zeros((D,), jnp.float32))
    hp = _linear(h, wpad, jnp.zeros((D,), jnp.float32))
    qn = qp[:, 0]
    pn = hp[:, 1]

    score = qn[col] + pn[row] + pool_params["att_b"][0]
    score = jax.nn.leaky_relu(score, negative_slope=0.2)
    smax = jax.ops.segment_max(score, col, num_segments=n)
    score = jnp.exp(score - smax[col])
    ssum = jax.ops.segment_sum(score, col, num_segments=n)
    score = score / (ssum[col] + 1e-16)

    score_p = _pad_to(score, ne_pad, 0.0)
    xn_acc = _sc_seg_sum(h.reshape(1, n, D), row_p,
                         _pad_to(col, ne_pad, _DUMMY), score_p)
    xn = xn_acc[0, 0, :n, :] + xn_acc[0, 1, :n, :]

    lepad = jnp.zeros((D, D), jnp.float32)
    lepad = (lepad.at[:, 0:1].set(pool_params["le1_W"])
                  .at[:, 1:2].set(pool_params["le2_W"])
                  .at[:, 2:3].set(pool_params["le3_W"]))
    lebias = jnp.zeros((D,), jnp.float32)
    lebias = lebias.at[0].set(pool_params["le1_b"][0]).at[2].set(pool_params["le3_b"][0])
    le = _linear(xn, lepad, lebias)
    a_n, b_n, c_n = le[:, 0], le[:, 1], le[:, 2]
    agg = jax.ops.segment_sum(a_n[row] - b_n[col], col, num_segments=n)
    fitness = jax.nn.sigmoid(agg + c_n)

    fit_k, perm = jax.lax.top_k(fitness, K_POOL)
    batch_p = batch[perm]

    pos = jnp.full((n,), -1, dtype=jnp.int32).at[perm].set(
        jnp.arange(K_POOL, dtype=jnp.int32))
    kcol = pos[col]
    valid = (kcol >= 0).astype(jnp.float32)
    kcol_safe = jnp.where(kcol >= 0, kcol, 0)
    kb = kcol_safe // D
    kd = kcol_safe % D
    S_r = jnp.zeros((8, _ACC, D), jnp.float32).at[kb, row, kd].add(score * valid)
    Mp = _sc_seg_sum(S_r, col_p, row_p)
    C = _coarsen_matmul(S_r, Mp)
    C = C * (1.0 - jnp.eye(KP, dtype=jnp.float32))

    perm_pad = _pad_to(perm, KP, 0)
    fit_pad = _pad_to(fit_k, KP, 0.0)
    g = xn[perm_pad] * fit_pad[:, None]
    for p in gnn_params:
        g = _pooled_gin(g, C, p)
    g = g[:K_POOL]

    sums = jnp.zeros((N_GRAPHS, D), jnp.float32).at[batch_p].add(g)
    cnt = jnp.zeros((N_GRAPHS,), jnp.float32).at[batch_p].add(1.0)
    out = sums / jnp.maximum(cnt, 1.0)[:, None]
    return out @ cls_params["W"] + cls_params["b"]


# speedup vs baseline: 1.9951x; 1.4434x over previous
import functools

import jax
import jax.numpy as jnp
from jax import lax
from jax.experimental import pallas as pl
from jax.experimental.pallas import tpu as pltpu
from jax.experimental.pallas import tpu_sc as plsc

N_NODES = 10000
D = 128
K_POOL = 1000
KP = 1024
N_GRAPHS = 128

_ACC = 10240
_TROWS = _ACC // 16
_DUMMY = 10100
_CH = 128


def _sc_body(nphase, nchunks, scaled, table, srcl, dstl, scalel, zrows, out,
             src_v0, src_v1, dst_v, scale_v, rows_v0, rows_v1, acc,
             sem0, sem1):
    cid = lax.axis_index("c")
    sid = lax.axis_index("s")
    eper = nchunks * _CH
    base = cid * (16 * eper) + sid * eper
    trow = pl.multiple_of(sid * _TROWS, 8)
    src_v = [src_v0, src_v1]
    rows_v = [rows_v0, rows_v1]
    sem = [sem0, sem1]
    for p in range(nphase):
        pltpu.sync_copy(zrows, acc.at[pl.ds(trow, _TROWS)])
        plsc.subcore_barrier()
        descs = [None, None]
        pltpu.sync_copy(srcl.at[pl.ds(pl.multiple_of(base, _CH), _CH)],
                        src_v[0])
        descs[0] = pltpu.async_copy(table.at[p].at[src_v[0]], rows_v[0],
                                    sem[0])
        for c in range(nchunks):
            buf = c % 2
            off = pl.multiple_of(base + c * _CH, _CH)
            descs[buf].wait()
            if c + 1 < nchunks:
                nb = (c + 1) % 2
                noff = pl.multiple_of(base + (c + 1) * _CH, _CH)
                pltpu.sync_copy(srcl.at[pl.ds(noff, _CH)], src_v[nb])
                descs[nb] = pltpu.async_copy(table.at[p].at[src_v[nb]],
                                             rows_v[nb], sem[nb])
            rv = rows_v[buf]
            pltpu.sync_copy(dstl.at[pl.ds(off, _CH)], dst_v)
            if scaled:
                pltpu.sync_copy(scalel.at[pl.ds(off, _CH)], scale_v)

                def mulrow(g, carry):
                    base16 = pl.multiple_of(g * 16, 16)
                    svec = scale_v[pl.ds(base16, 16)]
                    for i in range(16):
                        b = lax.gather(
                            svec, jnp.full((16, 1), i, jnp.int32),
                            lax.GatherDimensionNumbers(
                                offset_dims=(), collapsed_slice_dims=(0,),
                                start_index_map=(0,)),
                            (1,),
                            mode=lax.GatherScatterMode.PROMISE_IN_BOUNDS)
                        r = base16 + i
                        for j in range(8):
                            rv[r, pl.ds(j * 16, 16)] = (
                                rv[r, pl.ds(j * 16, 16)] * b)
                    return carry

                lax.fori_loop(0, _CH // 16, mulrow, 0)
            pltpu.sync_copy(rv, acc.at[dst_v], add=True)
        plsc.subcore_barrier()

        @pl.when(cid == 0)
        def _():
            pltpu.sync_copy(acc.at[pl.ds(trow, _TROWS)],
                            out.at[p, 0, pl.ds(trow, _TROWS)])

        @pl.when(cid == 1)
        def _():
            pltpu.sync_copy(acc.at[pl.ds(trow, _TROWS)],
                            out.at[p, 1, pl.ds(trow, _TROWS)])

        if p < nphase - 1:
            plsc.subcore_barrier()


def _sc_seg_sum(table, srcl, dstl, scalel=None):
    P = table.shape[0]
    EP = srcl.shape[0]
    nchunks = EP // (32 * _CH)
    scaled = scalel is not None
    if scalel is None:
        scalel = jnp.zeros((_CH,), jnp.float32)
    zrows = jnp.zeros((_TROWS, D), jnp.float32)
    mesh = plsc.VectorSubcoreMesh(core_axis_name="c", subcore_axis_name="s")
    f = pl.kernel(
        functools.partial(_sc_body, P, nchunks, scaled),
        out_type=jax.ShapeDtypeStruct((P, 2, _ACC, D), jnp.float32),
        mesh=mesh,
        scratch_types=[
            pltpu.VMEM((_CH,), jnp.int32),
            pltpu.VMEM((_CH,), jnp.int32),
            pltpu.VMEM((_CH,), jnp.int32),
            pltpu.VMEM((_CH,), jnp.float32),
            pltpu.VMEM((_CH, D), jnp.float32),
            pltpu.VMEM((_CH, D), jnp.float32),
            pltpu.VMEM_SHARED((_ACC, D), jnp.float32),
            pltpu.SemaphoreType.DMA,
            pltpu.SemaphoreType.DMA,
        ],
    )
    return f(table, srcl, dstl, scalel, zrows)


def _sc_edge_body(nchunks, mode, qt, pt, esc, coll, rowl, zvec, eout, psum,
                  cb0, cb1, rb0, rb1, qb0, qb1, pb0, pb1, ebuf, acc,
                  smq0, smq1, smp0, smp1):
    cid = lax.axis_index("c")
    sid = lax.axis_index("s")
    base = cid * (16 * nchunks) + sid * nchunks
    trow = pl.multiple_of(sid * 640, 8)
    cb = [cb0, cb1]
    rb = [rb0, rb1]
    qb = [qb0, qb1]
    pb = [pb0, pb1]
    smq = [smq0, smq1]
    smp = [smp0, smp1]
    use_q = mode in ("score", "norm")
    use_p = mode in ("score", "suma")
    use_esc = mode == "norm"
    emit = mode in ("score", "norm")
    scatter = mode != "norm"
    if scatter:
        pltpu.sync_copy(zvec, acc.at[pl.ds(trow, 640)])
        plsc.subcore_barrier()
    if mode == "deg":
        for j in range(8):
            ebuf[pl.ds(j * 16, 16)] = jnp.full((16,), 1.0, jnp.float32)
    dq = [None, None]
    dp = [None, None]

    def fire(c, b):
        off = pl.multiple_of((base + c) * _CH, _CH)
        pltpu.sync_copy(coll.at[pl.ds(off, _CH)], cb[b])
        if use_p:
            pltpu.sync_copy(rowl.at[pl.ds(off, _CH)], rb[b])
        if use_q:
            dq[b] = pltpu.async_copy(qt.at[cb[b]], qb[b], smq[b])
        if use_p:
            dp[b] = pltpu.async_copy(pt.at[rb[b]], pb[b], smp[b])
        if use_esc:
            dp[b] = pltpu.async_copy(esc.at[pl.ds(off, _CH)], pb[b], smp[b])

    fire(0, 0)
    for c in range(nchunks):
        buf = c % 2
        if use_q:
            dq[buf].wait()
        if use_p or use_esc:
            dp[buf].wait()
        if c + 1 < nchunks:
            fire(c + 1, (c + 1) % 2)
        if mode != "deg":
            for j in range(8):
                sl = pl.ds(j * 16, 16)
                if mode == "score":
                    s = qb[buf][sl] + pb[buf][sl]
                    s = jnp.exp(jnp.maximum(s, s * 0.2))
                elif mode == "norm":
                    s = qb[buf][sl] * pb[buf][sl]
                else:
                    s = pb[buf][sl]
                ebuf[sl] = s
        if emit:
            off = pl.multiple_of((base + c) * _CH, _CH)
            pltpu.sync_copy(ebuf, eout.at[pl.ds(off, _CH)])
        if scatter:
            pltpu.sync_copy(ebuf, acc.at[cb[buf]], add=True)
    if scatter:
        plsc.subcore_barrier()

        @pl.when(cid == 0)
        def _():
            pltpu.sync_copy(acc.at[pl.ds(trow, 640)],
                            psum.at[pl.ds(trow, 640)])

        @pl.when(cid == 1)
        def _():
            trow1 = pl.multiple_of(_ACC + sid * 640, 8)
            pltpu.sync_copy(acc.at[pl.ds(trow, 640)],
                            psum.at[pl.ds(trow1, 640)])


def _sc_edge_sum(mode, coll, rowl, qt=None, pt=None, esc=None):
    ep = coll.shape[0]
    nchunks = ep // (32 * _CH)
    zvec = jnp.zeros((640,), jnp.float32)
    dummy_t = jnp.zeros((8,), jnp.float32)
    dummy_e = jnp.zeros((_CH,), jnp.float32)
    qt = dummy_t if qt is None else qt
    pt = dummy_t if pt is None else pt
    esc = dummy_e if esc is None else esc
    mesh = plsc.VectorSubcoreMesh(core_axis_name="c", subcore_axis_name="s")
    outs = (jax.ShapeDtypeStruct((ep,), jnp.float32),
            jax.ShapeDtypeStruct((2 * _ACC,), jnp.float32))
    f = pl.kernel(
        functools.partial(_sc_edge_body, nchunks, mode),
        out_type=outs,
        mesh=mesh,
        scratch_types=[
            pltpu.VMEM((_CH,), jnp.int32),
            pltpu.VMEM((_CH,), jnp.int32),
            pltpu.VMEM((_CH,), jnp.int32),
            pltpu.VMEM((_CH,), jnp.int32),
            pltpu.VMEM((_CH,), jnp.float32),
            pltpu.VMEM((_CH,), jnp.float32),
            pltpu.VMEM((_CH,), jnp.float32),
            pltpu.VMEM((_CH,), jnp.float32),
            pltpu.VMEM((_CH,), jnp.float32),
            pltpu.VMEM_SHARED((_ACC,), jnp.float32),
            pltpu.SemaphoreType.DMA,
            pltpu.SemaphoreType.DMA,
            pltpu.SemaphoreType.DMA,
            pltpu.SemaphoreType.DMA,
        ],
    )
    return f(qt, pt, esc, coll, rowl, zvec)


_SFLAT = _ACC * D
_SDUMMY = 10100 * D


def _sc_s_body(nchunks, kcoll, rowl, esc, zvec, sp,
               kc0, kc1, rw0, rw1, eb0, eb1, vbuf, tbuf, acc, sme0, sme1):
    cid = lax.axis_index("c")
    sid = lax.axis_index("s")
    base = cid * (16 * nchunks) + sid * nchunks
    tflat = pl.multiple_of(sid * (_SFLAT // 16), 8)
    kcb = [kc0, kc1]
    rwb = [rw0, rw1]
    eb = [eb0, eb1]
    sme = [sme0, sme1]

    def fire(c, b):
        off = pl.multiple_of((base + c) * _CH, _CH)
        pltpu.sync_copy(kcoll.at[pl.ds(off, _CH)], kcb[b])
        pltpu.sync_copy(rowl.at[pl.ds(off, _CH)], rwb[b])
        pltpu.async_copy(esc.at[pl.ds(off, _CH)], eb[b], sme[b])

    def drain(b):
        off0 = pl.multiple_of(base * _CH, _CH)
        pltpu.make_async_copy(esc.at[pl.ds(off0, _CH)], eb[b], sme[b]).wait()

    for kb in range(8):
        pltpu.sync_copy(zvec, acc.at[pl.ds(tflat, _SFLAT // 16)])
        plsc.subcore_barrier()
        fire(0, 0)

        def compute(c, b):
            for j in range(8):
                sl = pl.ds(j * 16, 16)
                kcol = kcb[b][sl]
                rowv = rwb[b][sl]
                mask = (kcol >= 0) & ((kcol >> 7) == kb)
                tgt = jnp.where(
                    mask,
                    (rowv << 7) | (jnp.maximum(kcol, 0) & 127),
                    _SDUMMY)
                vbuf[sl] = eb[b][sl]
                tbuf[sl] = tgt
            pltpu.sync_copy(vbuf, acc.at[tbuf], add=True)

        def group(g, carry):
            c0 = 2 * g
            drain(0)
            fire(c0 + 1, 1)
            compute(c0, 0)
            drain(1)

            @pl.when(c0 + 2 < nchunks)
            def _():
                fire(c0 + 2, 0)

            compute(c0 + 1, 1)
            return carry

        lax.fori_loop(0, nchunks // 2, group, 0)
        plsc.subcore_barrier()

        @pl.when(cid == 0)
        def _():
            pltpu.sync_copy(acc.at[pl.ds(tflat, _SFLAT // 16)],
                            sp.at[0, kb, pl.ds(tflat, _SFLAT // 16)])

        @pl.when(cid == 1)
        def _():
            pltpu.sync_copy(acc.at[pl.ds(tflat, _SFLAT // 16)],
                            sp.at[1, kb, pl.ds(tflat, _SFLAT // 16)])

        if kb < 7:
            plsc.subcore_barrier()


def _sc_s_scatter(kcoll, rowl, esc):
    ep = kcoll.shape[0]
    nchunks = ep // (32 * _CH)
    zvec = jnp.zeros((_SFLAT // 16,), jnp.float32)
    mesh = plsc.VectorSubcoreMesh(core_axis_name="c", subcore_axis_name="s")
    f = pl.kernel(
        functools.partial(_sc_s_body, nchunks),
        out_type=jax.ShapeDtypeStruct((2, 8, _SFLAT), jnp.float32),
        mesh=mesh,
        scratch_types=[
            pltpu.VMEM((_CH,), jnp.int32),
            pltpu.VMEM((_CH,), jnp.int32),
            pltpu.VMEM((_CH,), jnp.int32),
            pltpu.VMEM((_CH,), jnp.int32),
            pltpu.VMEM((_CH,), jnp.float32),
            pltpu.VMEM((_CH,), jnp.float32),
            pltpu.VMEM((_CH,), jnp.float32),
            pltpu.VMEM((_CH,), jnp.int32),
            pltpu.VMEM_SHARED((_SFLAT,), jnp.float32),
            pltpu.SemaphoreType.DMA,
            pltpu.SemaphoreType.DMA,
        ],
    )
    return f(kcoll, rowl, esc, zvec)


def _gin_mlp_body(h_ref, a_ref, w1_ref, b1_ref, w2_ref, b2_ref, o_ref):
    z = h_ref[...] + a_ref[0] + a_ref[1]
    z = jnp.maximum(jnp.dot(z, w1_ref[...],
                            preferred_element_type=jnp.float32) + b1_ref[...], 0.0)
    z = jnp.dot(z, w2_ref[...], preferred_element_type=jnp.float32) + b2_ref[...]
    o_ref[...] = jnp.maximum(z, 0.0)


def _gin_mlp(h, pacc, p):
    n = h.shape[0]
    blk = 1000
    return pl.pallas_call(
        _gin_mlp_body,
        grid=(n // blk,),
        in_specs=[
            pl.BlockSpec((blk, D), lambda i: (i, 0)),
            pl.BlockSpec((2, blk, D), lambda i: (0, i, 0)),
            pl.BlockSpec((D, D), lambda i: (0, 0)),
            pl.BlockSpec((1, D), lambda i: (0, 0)),
            pl.BlockSpec((D, D), lambda i: (0, 0)),
            pl.BlockSpec((1, D), lambda i: (0, 0)),
        ],
        out_specs=pl.BlockSpec((blk, D), lambda i: (i, 0)),
        out_shape=jax.ShapeDtypeStruct((n, D), jnp.float32),
    )(h, pacc, p["W1"], p["b1"].reshape(1, D), p["W2"], p["b2"].reshape(1, D))


def _linear_body(x_ref, w_ref, b_ref, o_ref):
    o_ref[...] = jnp.dot(x_ref[...], w_ref[...],
                         preferred_element_type=jnp.float32) + b_ref[...]


def _linear(x, w, b):
    n, dout = x.shape[0], w.shape[1]
    blk = 1000 if n % 1000 == 0 else n
    return pl.pallas_call(
        _linear_body,
        grid=(n // blk,),
        in_specs=[
            pl.BlockSpec((blk, D), lambda i: (i, 0)),
            pl.BlockSpec((D, dout), lambda i: (0, 0)),
            pl.BlockSpec((1, dout), lambda i: (0, 0)),
        ],
        out_specs=pl.BlockSpec((blk, dout), lambda i: (i, 0)),
        out_shape=jax.ShapeDtypeStruct((n, dout), jnp.float32),
    )(x, w, b.reshape(1, dout))


def _cmat_body(s_ref, m_ref, o_ref):
    @pl.when(pl.program_id(0) == 0)
    def _init():
        o_ref[...] = jnp.zeros_like(o_ref)

    s = s_ref[...]
    m = m_ref[...]
    for kb2 in range(8):
        msum = m[kb2, 0] + m[kb2, 1]
        prod = lax.dot_general(s, msum, (((1,), (0,)), ((), ())),
                               preferred_element_type=jnp.float32)
        o_ref[:, :, kb2, :] += prod


def _coarsen_matmul(S_r, Mp):
    blk = 1024
    C = pl.pallas_call(
        _cmat_body,
        grid=(_ACC // blk,),
        in_specs=[
            pl.BlockSpec((8, blk, D), lambda i: (0, i, 0)),
            pl.BlockSpec((8, 2, blk, D), lambda i: (0, 0, i, 0)),
        ],
        out_specs=pl.BlockSpec((8, D, 8, D), lambda i: (0, 0, 0, 0)),
        out_shape=jax.ShapeDtypeStruct((8, D, 8, D), jnp.float32),
    )(S_r, Mp)
    return C.reshape(KP, KP)


def _pooled_gin_body(g_ref, c_ref, w1_ref, b1_ref, w2_ref, b2_ref, o_ref):
    g = g_ref[...]
    agg = lax.dot_general(c_ref[...], g, (((0,), (0,)), ((), ())),
                          preferred_element_type=jnp.float32)
    z = g + agg
    z = jnp.maximum(jnp.dot(z, w1_ref[...],
                            preferred_element_type=jnp.float32) + b1_ref[...], 0.0)
    z = jnp.dot(z, w2_ref[...], preferred_element_type=jnp.float32) + b2_ref[...]
    o_ref[...] = jnp.maximum(z, 0.0)


def _pooled_gin(g, C, p):
    k = g.shape[0]
    return pl.pallas_call(
        _pooled_gin_body,
        in_specs=[
            pl.BlockSpec((k, D), lambda: (0, 0)),
            pl.BlockSpec((k, k), lambda: (0, 0)),
            pl.BlockSpec((D, D), lambda: (0, 0)),
            pl.BlockSpec((1, D), lambda: (0, 0)),
            pl.BlockSpec((D, D), lambda: (0, 0)),
            pl.BlockSpec((1, D), lambda: (0, 0)),
        ],
        out_specs=pl.BlockSpec((k, D), lambda: (0, 0)),
        out_shape=jax.ShapeDtypeStruct((k, D), jnp.float32),
    )(g, C, p["W1"], p["b1"].reshape(1, D), p["W2"], p["b2"].reshape(1, D))


def _pad_to(a, size, fill):
    return jnp.concatenate(
        [a, jnp.full((size - a.shape[0],), fill, a.dtype)])


def kernel(x, edge_index, batch, sub_params, pool_params, gnn_params, cls_params):
    n = x.shape[0]
    src, dst = edge_index[0], edge_index[1]
    ne_pad_gin = 163840
    ne_pad = 172032

    src_p = _pad_to(src, ne_pad_gin, 0)
    dst_p = _pad_to(dst, ne_pad_gin, _DUMMY)

    h = x
    for p in sub_params:
        pacc = _sc_seg_sum(h.reshape(1, n, D), src_p, dst_p)
        h = _gin_mlp(h, pacc[0, :, :n, :], p)

    loop = jnp.arange(n, dtype=src.dtype)
    row = jnp.concatenate([src, loop])
    col = jnp.concatenate([dst, loop])
    row_p = _pad_to(row, ne_pad, 0)
    col_p = _pad_to(col, ne_pad, _DUMMY)

    xq = jax.ops.segment_max(h[row], col, num_segments=n)
    xq_lin = _linear(xq, pool_params["lin_W"], pool_params["lin_b"])

    att_w1 = pool_params["att_W"][:D]
    att_w2 = pool_params["att_W"][D:]
    wpad = jnp.zeros((D, D), jnp.float32)
    wpad = wpad.at[:, 0:1].set(att_w1).at[:, 1:2].set(att_w2)
    qp = _linear(xq_lin, wpad, jnp.zeros((D,), jnp.float32))
    hp = _linear(h, wpad, jnp.zeros((D,), jnp.float32))
    qn_pad = _pad_to(qp[:, 0] + pool_params["att_b"][0], _ACC, 0.0)
    pn_pad = _pad_to(hp[:, 1], _ACC, 0.0)

    escore, ssum_p = _sc_edge_sum("score", col_p, row_p,
                                  qt=qn_pad, pt=pn_pad)
    ssum_p = ssum_p.reshape(2, _ACC)
    inv = 1.0 / (ssum_p[0] + ssum_p[1] + 1e-16)
    score_p, _ = _sc_edge_sum("norm", col_p, row_p, qt=inv, esc=escore)

    xn_acc = _sc_seg_sum(h.reshape(1, n, D), row_p,
                         _pad_to(col, ne_pad, _DUMMY), score_p)
    xn = xn_acc[0, 0, :n, :] + xn_acc[0, 1, :n, :]

    lepad = jnp.zeros((D, D), jnp.float32)
    lepad = (lepad.at[:, 0:1].set(pool_params["le1_W"])
                  .at[:, 1:2].set(pool_params["le2_W"])
                  .at[:, 2:3].set(pool_params["le3_W"]))
    lebias = jnp.zeros((D,), jnp.float32)
    lebias = lebias.at[0].set(pool_params["le1_b"][0]).at[2].set(pool_params["le3_b"][0])
    le = _linear(xn, lepad, lebias)
    a_n, b_n, c_n = le[:, 0], le[:, 1], le[:, 2]
    a_pad = _pad_to(a_n, _ACC, 0.0)
    _, asum_p = _sc_edge_sum("suma", col_p, row_p, pt=a_pad)
    asum = asum_p.reshape(2, _ACC)[0] + asum_p.reshape(2, _ACC)[1]
    _, deg_p = _sc_edge_sum("deg", col_p, row_p)
    deg = deg_p.reshape(2, _ACC)[0] + deg_p.reshape(2, _ACC)[1]
    agg = asum[:n] - b_n * deg[:n]
    fitness = jax.nn.sigmoid(agg + c_n)

    fit_k, perm = jax.lax.top_k(fitness, K_POOL)
    batch_p = batch[perm]

    pos_pad = jnp.full((_ACC,), -1, dtype=jnp.int32).at[perm].set(
        jnp.arange(K_POOL, dtype=jnp.int32))
    kcol_p = pos_pad[col_p]
    sp = _sc_s_scatter(kcol_p, row_p, score_p)
    S_r = (sp[0] + sp[1]).reshape(8, _ACC, D)
    Mp = _sc_seg_sum(S_r, col_p, row_p)
    C = _coarsen_matmul(S_r, Mp)
    C = C * (1.0 - jnp.eye(KP, dtype=jnp.float32))

    perm_pad = _pad_to(perm, KP, 0)
    fit_pad = _pad_to(fit_k, KP, 0.0)
    g = xn[perm_pad] * fit_pad[:, None]
    for p in gnn_params:
        g = _pooled_gin(g, C, p)
    g = g[:K_POOL]

    sums = jnp.zeros((N_GRAPHS, D), jnp.float32).at[batch_p].add(g)
    cnt = jnp.zeros((N_GRAPHS,), jnp.float32).at[batch_p].add(1.0)
    out = sums / jnp.maximum(cnt, 1.0)[:, None]
    return out @ cls_params["W"] + cls_params["b"]
